# baseline (speedup 1.0000x reference)
import functools

import jax
import jax.numpy as jnp
from jax import lax
from jax.experimental import pallas as pl
from jax.experimental.pallas import tpu as pltpu
from jax.experimental.pallas import tpu_sc as plsc

H = 512
W = 512
HW = H * W
NPTS = 100000
SCALE = 512.0 / 511.0

_NC = 2
_NS = 16
_NW = _NC * _NS
_BPW = 3136
_LAST = NPTS - _BPW
_NCH = 4
_CPTS = _BPW // _NCH
_L = 16
_U = 1


@functools.partial(
    pl.kernel,
    mesh=plsc.VectorSubcoreMesh(core_axis_name="c", subcore_axis_name="s"),
    out_type=jax.ShapeDtypeStruct((2, NPTS), jnp.float32),
    compiler_params=pltpu.CompilerParams(use_tc_tiling_on_sc=False),
    scratch_types=[
        pltpu.VMEM((_BPW,), jnp.float32),
        pltpu.VMEM((_BPW,), jnp.float32),
        pltpu.VMEM((_CPTS,), jnp.int32),
        pltpu.VMEM((_CPTS,), jnp.int32),
        pltpu.VMEM((_CPTS,), jnp.int32),
        pltpu.VMEM((_CPTS,), jnp.int32),
        pltpu.VMEM((_CPTS,), jnp.float32),
        pltpu.VMEM((_CPTS,), jnp.float32),
        pltpu.VMEM((_CPTS,), jnp.float32),
        pltpu.VMEM((_CPTS,), jnp.float32),
        pltpu.VMEM((_CPTS,), jnp.float32),
        pltpu.VMEM((_CPTS,), jnp.float32),
        pltpu.VMEM((_CPTS,), jnp.float32),
        pltpu.VMEM((_CPTS,), jnp.float32),
        pltpu.VMEM((_BPW,), jnp.float32),
        pltpu.VMEM((_BPW,), jnp.float32),
        pltpu.VMEM_SHARED((HW,), jnp.float32),
        pltpu.VMEM_SHARED((HW,), jnp.float32),
        pltpu.SemaphoreType.DMA,
        pltpu.SemaphoreType.DMA,
        pltpu.SemaphoreType.DMA,
        pltpu.SemaphoreType.DMA,
        pltpu.SemaphoreType.DMA,
        pltpu.SemaphoreType.DMA,
    ],
)
def _sc_points(pt_hbm, fl_hbm, o_hbm,
               px_v, py_v, i0_v, i1_v, i2_v, i3_v,
               a0_v, a1_v, a2_v, a3_v, b0_v, b1_v, b2_v, b3_v,
               o0_v, o1_v, sf0_v, sf1_v, sem0, sem1, sem2, sem3, ssem0, ssem1):
    idx_refs = (i0_v, i1_v, i2_v, i3_v)
    ga_refs = (a0_v, a1_v, a2_v, a3_v)
    gb_refs = (b0_v, b1_v, b2_v, b3_v)
    sems = (sem0, sem1, sem2, sem3)

    sub = lax.axis_index("s")
    wid = sub * _NC + lax.axis_index("c")
    base = jnp.minimum(wid * _BPW, _LAST)

    sz = HW // _NS
    st0 = pltpu.async_copy(fl_hbm.at[0, pl.ds(sub * sz, sz)], sf0_v.at[pl.ds(sub * sz, sz)], ssem0)
    st1 = pltpu.async_copy(fl_hbm.at[1, pl.ds(sub * sz, sz)], sf1_v.at[pl.ds(sub * sz, sz)], ssem1)

    pltpu.sync_copy(pt_hbm.at[0, pl.ds(base, _BPW)], px_v)
    pltpu.sync_copy(pt_hbm.at[1, pl.ds(base, _BPW)], py_v)

    def make_idx_body(ch):
        def body(i, carry):
            for u in range(_U):
                s = pl.ds(ch * _CPTS + (i * _U + u) * _L, _L)
                xi = jnp.minimum((px_v[s] + 0.5).astype(jnp.int32), H - 1)
                yi = jnp.minimum((py_v[s] + 0.5).astype(jnp.int32), W - 1)
                idx_refs[ch][pl.ds((i * _U + u) * _L, _L)] = xi * W + yi
            return carry
        return body

    for ch in range(_NCH):
        lax.fori_loop(0, _CPTS // (_L * _U), make_idx_body(ch), 0)

    st0.wait()
    st1.wait()
    plsc.subcore_barrier()

    copies = []
    for ch in range(_NCH):
        copies.append(pltpu.async_copy(sf0_v.at[idx_refs[ch]], ga_refs[ch], sems[ch]))
        copies.append(pltpu.async_copy(sf1_v.at[idx_refs[ch]], gb_refs[ch], sems[ch]))

    def make_out_body(ch):
        def body(i, carry):
            for u in range(_U):
                s = pl.ds(ch * _CPTS + (i * _U + u) * _L, _L)
                cs = pl.ds((i * _U + u) * _L, _L)
                xi = jnp.minimum((px_v[s] + 0.5).astype(jnp.int32), H - 1)
                yi = jnp.minimum((py_v[s] + 0.5).astype(jnp.int32), W - 1)
                o0_v[s] = (xi.astype(jnp.float32) + ga_refs[ch][cs]) * SCALE
                o1_v[s] = (yi.astype(jnp.float32) + gb_refs[ch][cs]) * SCALE
            return carry
        return body

    for ch in range(_NCH):
        copies[2 * ch].wait()
        copies[2 * ch + 1].wait()
        lax.fori_loop(0, _CPTS // (_L * _U), make_out_body(ch), 0)

    pltpu.sync_copy(o0_v, o_hbm.at[0, pl.ds(base, _BPW)])
    pltpu.sync_copy(o1_v, o_hbm.at[1, pl.ds(base, _BPW)])


def kernel(point, flow, grid):
    del grid
    o = _sc_points(point[0].T, flow.reshape(2, HW))
    return o.T[None]


# speedup vs baseline: 1.0968x; 1.0968x over previous
import functools

import jax
import jax.numpy as jnp
from jax import lax
from jax.experimental import pallas as pl
from jax.experimental.pallas import tpu as pltpu
from jax.experimental.pallas import tpu_sc as plsc

H = 512
W = 512
HW = H * W
NPTS = 100000
SCALE = 512.0 / 511.0

_NC = 2
_NS = 16
_NW = _NC * _NS
_BPW = 3136
_LAST = NPTS - _BPW
_NCH = 4
_CPTS = _BPW // _NCH
_L = 16
_U = 1


@functools.partial(
    pl.kernel,
    mesh=plsc.VectorSubcoreMesh(core_axis_name="c", subcore_axis_name="s"),
    out_type=jax.ShapeDtypeStruct((2, NPTS), jnp.float32),
    compiler_params=pltpu.CompilerParams(use_tc_tiling_on_sc=False),
    scratch_types=[
        pltpu.VMEM((_BPW,), jnp.float32),
        pltpu.VMEM((_BPW,), jnp.float32),
        pltpu.VMEM((_CPTS,), jnp.int32),
        pltpu.VMEM((_CPTS,), jnp.int32),
        pltpu.VMEM((_CPTS,), jnp.int32),
        pltpu.VMEM((_CPTS,), jnp.int32),
        pltpu.VMEM((_CPTS,), jnp.float32),
        pltpu.VMEM((_CPTS,), jnp.float32),
        pltpu.VMEM((_CPTS,), jnp.float32),
        pltpu.VMEM((_CPTS,), jnp.float32),
        pltpu.VMEM((_CPTS,), jnp.float32),
        pltpu.VMEM((_CPTS,), jnp.float32),
        pltpu.VMEM((_CPTS,), jnp.float32),
        pltpu.VMEM((_CPTS,), jnp.float32),
        pltpu.VMEM((_BPW,), jnp.float32),
        pltpu.VMEM((_BPW,), jnp.float32),
        pltpu.VMEM_SHARED((HW,), jnp.float32),
        pltpu.VMEM_SHARED((HW,), jnp.float32),
        pltpu.SemaphoreType.DMA,
        pltpu.SemaphoreType.DMA,
        pltpu.SemaphoreType.DMA,
        pltpu.SemaphoreType.DMA,
        pltpu.SemaphoreType.DMA,
        pltpu.SemaphoreType.DMA,
    ],
)
def _sc_points(pt_hbm, fl5_hbm, o_hbm,
               px_v, py_v, i0_v, i1_v, i2_v, i3_v,
               a0_v, a1_v, a2_v, a3_v, b0_v, b1_v, b2_v, b3_v,
               o0_v, o1_v, sf0_v, sf1_v, sem0, sem1, sem2, sem3, ssem0, ssem1):
    idx_refs = (i0_v, i1_v, i2_v, i3_v)
    ga_refs = (a0_v, a1_v, a2_v, a3_v)
    gb_refs = (b0_v, b1_v, b2_v, b3_v)
    sems = (sem0, sem1, sem2, sem3)

    sub = lax.axis_index("s")
    wid = sub * _NC + lax.axis_index("c")
    base = jnp.minimum(wid * _BPW, _LAST)

    sz = HW // _NS
    st0 = pltpu.async_copy(fl5_hbm.at[0, pl.ds(sub * sz, sz)], sf0_v.at[pl.ds(sub * sz, sz)], ssem0)
    st1 = pltpu.async_copy(fl5_hbm.at[1, pl.ds(sub * sz, sz)], sf1_v.at[pl.ds(sub * sz, sz)], ssem1)

    pltpu.sync_copy(pt_hbm.at[0, pl.ds(base, _BPW)], px_v)
    pltpu.sync_copy(pt_hbm.at[1, pl.ds(base, _BPW)], py_v)

    def make_idx_body(ch):
        def body(i, carry):
            for u in range(_U):
                s = pl.ds(ch * _CPTS + (i * _U + u) * _L, _L)
                xi = jnp.minimum((px_v[s] + 0.5).astype(jnp.int32), H - 1)
                yi = jnp.minimum((py_v[s] + 0.5).astype(jnp.int32), W - 1)
                t = (((xi >> 3) * 4 + (yi >> 7)) * 1024
                     + (xi & 7) * 128 + (yi & 127))
                idx_refs[ch][pl.ds((i * _U + u) * _L, _L)] = t
            return carry
        return body

    for ch in range(_NCH):
        lax.fori_loop(0, _CPTS // (_L * _U), make_idx_body(ch), 0)

    st0.wait()
    st1.wait()
    plsc.subcore_barrier()

    copies = []
    for ch in range(_NCH):
        copies.append(pltpu.async_copy(sf0_v.at[idx_refs[ch]], ga_refs[ch], sems[ch]))
        copies.append(pltpu.async_copy(sf1_v.at[idx_refs[ch]], gb_refs[ch], sems[ch]))

    def make_out_body(ch):
        def body(i, carry):
            for u in range(_U):
                s = pl.ds(ch * _CPTS + (i * _U + u) * _L, _L)
                cs = pl.ds((i * _U + u) * _L, _L)
                xi = jnp.minimum((px_v[s] + 0.5).astype(jnp.int32), H - 1)
                yi = jnp.minimum((py_v[s] + 0.5).astype(jnp.int32), W - 1)
                o0_v[s] = (xi.astype(jnp.float32) + ga_refs[ch][cs]) * SCALE
                o1_v[s] = (yi.astype(jnp.float32) + gb_refs[ch][cs]) * SCALE
            return carry
        return body

    for ch in range(_NCH):
        copies[2 * ch].wait()
        copies[2 * ch + 1].wait()
        lax.fori_loop(0, _CPTS // (_L * _U), make_out_body(ch), 0)

    pltpu.sync_copy(o0_v, o_hbm.at[0, pl.ds(base, _BPW)])
    pltpu.sync_copy(o1_v, o_hbm.at[1, pl.ds(base, _BPW)])


def kernel(point, flow, grid):
    del grid
    fl2 = flow[0].reshape(2, 64, 8, 4, 128).transpose(0, 1, 3, 2, 4).reshape(2, HW)
    o = _sc_points(point[0].T, fl2)
    return o.T[None]
